# baseline (speedup 1.0000x reference)
import jax
import jax.numpy as jnp
from jax.experimental import pallas as pl
from jax.experimental.pallas import tpu as pltpu

H = 3
K = 16
OFFSET_SCALE = 10.0
T = 256
NCHUNK = 512

INTERPRET = False


def _gelu(x):
    return x * 0.5 * (1.0 + jax.lax.erf(x * (2.0 ** -0.5)))


def _fused_body(coords_ref, coordsT_ref, x_ref,
                wq_ref, bq_ref, wk_ref, bk_ref, wv_ref, bv_ref,
                w1_ref, b1_ref, w2d_ref, b2d_ref,
                p1_ref, p1b_ref, p2_ref, p2b_ref,
                wp_ref, bp_ref,
                out_ref, k_scr, v_scr):
    t = pl.program_id(1)
    N = x_ref.shape[1]
    C = x_ref.shape[2]
    D = C // H
    scale = D ** -0.5
    nch = N // NCHUNK

    @pl.when(t == 0)
    def _():
        xf = x_ref[0]
        k_scr[...] = (xf @ wk_ref[...] + bk_ref[...]).astype(k_scr.dtype)
        v_scr[...] = (xf @ wv_ref[...] + bv_ref[...]).astype(v_scr.dtype)

    xt = x_ref[0, pl.ds(t * T, T), :]
    q = xt @ wq_ref[...] + bq_ref[...]
    hid = _gelu(xt @ w1_ref[...] + b1_ref[...])

    offs = []
    sps = []
    for d in range(3):
        od = hid @ w2d_ref[d] + b2d_ref[d]
        cd = coords_ref[0, pl.ds(t * T, T), d:d + 1]
        offs.append(od)
        sps.append(cd + OFFSET_SCALE * od)

    c0 = coordsT_ref[0, 0:1, :]
    c1 = coordsT_ref[0, 1:2, :]
    c2 = coordsT_ref[0, 2:3, :]
    cn2 = c0 * c0 + c1 * c1 + c2 * c2
    s3 = cn2[None] + (-2.0 * sps[0])[:, :, None] * c0[None]
    s3 = s3 + (-2.0 * sps[1])[:, :, None] * c1[None]
    s3 = s3 + (-2.0 * sps[2])[:, :, None] * c2[None]
    ridx = jnp.argmin(s3, axis=-1).astype(jnp.int32)

    iota = jax.lax.broadcasted_iota(jnp.int32, (1, 1, N), 2)
    oh2 = (ridx[:, :, None] == iota).astype(jnp.bfloat16).reshape(T * K, N)
    sk = jnp.dot(oh2, k_scr[...], preferred_element_type=jnp.float32)
    sv = jnp.dot(oh2, v_scr[...], preferred_element_type=jnp.float32)

    sk3 = sk.reshape(T, K, C)
    sv3 = sv.reshape(T, K, C)
    prod = q[:, None, :] * sk3

    ph = (offs[0][:, :, None] * p1_ref[0][None] +
          offs[1][:, :, None] * p1_ref[1][None] +
          offs[2][:, :, None] * p1_ref[2][None] + p1b_ref[...])
    ph = _gelu(ph)

    ihh = (jax.lax.broadcasted_iota(jnp.int32, (C, H), 0) // D ==
           jax.lax.broadcasted_iota(jnp.int32, (C, H), 1)
           ).astype(jnp.float32) * scale
    scores = jax.lax.dot_general(prod, ihh, (((2,), (0,)), ((), ())),
                                 preferred_element_type=jnp.float32)
    bias = jax.lax.dot_general(ph, p2_ref[...], (((2,), (0,)), ((), ())),
                               preferred_element_type=jnp.float32)
    st = scores + bias + p2b_ref[...]

    outs = []
    for h in range(H):
        s_h = st[:, :, h:h + 1]
        m = jnp.max(s_h, axis=1, keepdims=True)
        e = jnp.exp(s_h - m)
        w_h = e / jnp.sum(e, axis=1, keepdims=True)
        o_h = jnp.sum(sv3[:, :, h * D:(h + 1) * D] * w_h, axis=1)
        outs.append(o_h)

    out = jnp.concatenate(outs, axis=-1)
    out_ref[0] = out @ wp_ref[...] + bp_ref[...]


def kernel(coords, x, qkv_w, qkv_b, proj_w, proj_b, off1_w, off1_b,
           off2_w, off2_b, pos1_w, pos1_b, pos2_w, pos2_b):
    B, N, C = x.shape
    C2 = off1_w.shape[0]
    PH = pos1_w.shape[0]

    wq = qkv_w[0:C].T
    wk = qkv_w[C:2 * C].T
    wv = qkv_w[2 * C:3 * C].T
    bq = qkv_b[0:C][None]
    bk = qkv_b[C:2 * C][None]
    bv = qkv_b[2 * C:3 * C][None]
    w2d = jnp.stack([off2_w[d::3].T for d in range(3)])
    b2d = jnp.stack([off2_b[d::3][None] for d in range(3)])
    coordsT = coords.transpose(0, 2, 1)
    p1 = pos1_w.T.reshape(3, 1, PH)
    p1b = pos1_b.reshape(1, 1, PH)
    p2 = pos2_w.T
    p2b = pos2_b.reshape(1, 1, H)
    wp = proj_w.T
    bp = proj_b[None]

    grid = (B, N // T)

    def full(arr):
        return pl.BlockSpec(arr.shape, lambda b, t: (0,) * arr.ndim)

    out = pl.pallas_call(
        _fused_body,
        grid=grid,
        in_specs=[
            pl.BlockSpec((1, N, 3), lambda b, t: (b, 0, 0)),
            pl.BlockSpec((1, 3, N), lambda b, t: (b, 0, 0)),
            pl.BlockSpec((1, N, C), lambda b, t: (b, 0, 0)),
            full(wq), full(bq), full(wk), full(bk), full(wv), full(bv),
            full(off1_w.T), full(off1_b[None]), full(w2d), full(b2d),
            full(p1), full(p1b), full(p2), full(p2b),
            full(wp), full(bp),
        ],
        out_specs=pl.BlockSpec((1, T, C), lambda b, t: (b, t, 0)),
        out_shape=jax.ShapeDtypeStruct((B, N, C), jnp.float32),
        scratch_shapes=[
            pltpu.VMEM((N, C), jnp.bfloat16),
            pltpu.VMEM((N, C), jnp.bfloat16),
        ],
        compiler_params=pltpu.CompilerParams(
            dimension_semantics=("arbitrary", "arbitrary"),
        ),
        interpret=INTERPRET,
    )(coords, coordsT, x,
      wq, bq, wk, bk, wv, bv,
      off1_w.T, off1_b[None], w2d, b2d,
      p1, p1b, p2, p2b,
      wp, bp)
    return out


# speedup vs baseline: 1.0030x; 1.0030x over previous
import jax
import jax.numpy as jnp
from jax.experimental import pallas as pl
from jax.experimental.pallas import tpu as pltpu

H = 3
K = 16
OFFSET_SCALE = 10.0
T = 256
NCHUNK = 512


def _gelu(x):
    return x * 0.5 * (1.0 + jax.lax.erf(x * (2.0 ** -0.5)))


def _fused_body(coords_ref, coordsT_ref, x_ref,
                wq_ref, bq_ref, wk_ref, bk_ref, wv_ref, bv_ref,
                w1_ref, b1_ref, w2d_ref, b2d_ref,
                p1_ref, p1b_ref, p2_ref, p2b_ref,
                wp_ref, bp_ref,
                out_ref, k_scr, v_scr):
    t = pl.program_id(1)
    N = x_ref.shape[1]
    C = x_ref.shape[2]
    D = C // H
    scale = D ** -0.5
    nch = N // NCHUNK

    @pl.when(t == 0)
    def _():
        xf = x_ref[0]
        k_scr[...] = (xf @ wk_ref[...] + bk_ref[...]).astype(k_scr.dtype)
        v_scr[...] = (xf @ wv_ref[...] + bv_ref[...]).astype(v_scr.dtype)

    xt = x_ref[0, pl.ds(t * T, T), :]
    q = xt @ wq_ref[...] + bq_ref[...]
    hid = _gelu(xt @ w1_ref[...] + b1_ref[...])

    offs = []
    sps = []
    for d in range(3):
        od = hid @ w2d_ref[d] + b2d_ref[d]
        cd = coords_ref[0, pl.ds(t * T, T), d:d + 1]
        offs.append(od)
        sps.append(cd + OFFSET_SCALE * od)

    c0 = coordsT_ref[0, 0:1, :]
    c1 = coordsT_ref[0, 1:2, :]
    c2 = coordsT_ref[0, 2:3, :]
    cn2 = c0 * c0 + c1 * c1 + c2 * c2
    s3 = cn2[None] + (-2.0 * sps[0])[:, :, None] * c0[None]
    s3 = s3 + (-2.0 * sps[1])[:, :, None] * c1[None]
    s3 = s3 + (-2.0 * sps[2])[:, :, None] * c2[None]
    ridx = jnp.argmin(s3, axis=-1).astype(jnp.int32)

    sk = jnp.zeros((T * K, C), dtype=jnp.float32)
    sv = jnp.zeros((T * K, C), dtype=jnp.float32)
    iota = jax.lax.broadcasted_iota(jnp.int32, (1, 1, NCHUNK), 2)
    for c in range(nch):
        oh = (ridx[:, :, None] == (iota + c * NCHUNK)).astype(jnp.bfloat16)
        oh2 = oh.reshape(T * K, NCHUNK)
        kc = k_scr[pl.ds(c * NCHUNK, NCHUNK), :]
        vc = v_scr[pl.ds(c * NCHUNK, NCHUNK), :]
        sk = sk + jnp.dot(oh2, kc, preferred_element_type=jnp.float32)
        sv = sv + jnp.dot(oh2, vc, preferred_element_type=jnp.float32)

    sk3 = sk.reshape(T, K, C)
    sv3 = sv.reshape(T, K, C)
    prod = q[:, None, :] * sk3

    ph = (offs[0][:, :, None] * p1_ref[0][None] +
          offs[1][:, :, None] * p1_ref[1][None] +
          offs[2][:, :, None] * p1_ref[2][None] + p1b_ref[...])
    ph = _gelu(ph)

    ihh = (jax.lax.broadcasted_iota(jnp.int32, (C, H), 0) // D ==
           jax.lax.broadcasted_iota(jnp.int32, (C, H), 1)
           ).astype(jnp.float32) * scale
    scores = jax.lax.dot_general(prod, ihh, (((2,), (0,)), ((), ())),
                                 preferred_element_type=jnp.float32)
    bias = jax.lax.dot_general(ph, p2_ref[...], (((2,), (0,)), ((), ())),
                               preferred_element_type=jnp.float32)
    st = scores + bias + p2b_ref[...]

    outs = []
    for h in range(H):
        s_h = st[:, :, h:h + 1]
        m = jnp.max(s_h, axis=1, keepdims=True)
        e = jnp.exp(s_h - m)
        w_h = e / jnp.sum(e, axis=1, keepdims=True)
        o_h = jnp.sum(sv3[:, :, h * D:(h + 1) * D] * w_h, axis=1)
        outs.append(o_h)

    out = jnp.concatenate(outs, axis=-1)
    out_ref[0] = out @ wp_ref[...] + bp_ref[...]


def kernel(coords, x, qkv_w, qkv_b, proj_w, proj_b, off1_w, off1_b,
           off2_w, off2_b, pos1_w, pos1_b, pos2_w, pos2_b):
    B, N, C = x.shape
    PH = pos1_w.shape[0]

    wq = qkv_w[0:C].T
    wk = qkv_w[C:2 * C].T
    wv = qkv_w[2 * C:3 * C].T
    bq = qkv_b[0:C][None]
    bk = qkv_b[C:2 * C][None]
    bv = qkv_b[2 * C:3 * C][None]
    w2d = jnp.stack([off2_w[d::3].T for d in range(3)])
    b2d = jnp.stack([off2_b[d::3][None] for d in range(3)])
    coordsT = coords.transpose(0, 2, 1)
    p1 = pos1_w.T.reshape(3, 1, PH)
    p1b = pos1_b.reshape(1, 1, PH)
    p2 = pos2_w.T
    p2b = pos2_b.reshape(1, 1, H)
    wp = proj_w.T
    bp = proj_b[None]

    grid = (B, N // T)

    def full(arr):
        return pl.BlockSpec(arr.shape, lambda b, t: (0,) * arr.ndim)

    out = pl.pallas_call(
        _fused_body,
        grid=grid,
        in_specs=[
            pl.BlockSpec((1, N, 3), lambda b, t: (b, 0, 0)),
            pl.BlockSpec((1, 3, N), lambda b, t: (b, 0, 0)),
            pl.BlockSpec((1, N, C), lambda b, t: (b, 0, 0)),
            full(wq), full(bq), full(wk), full(bk), full(wv), full(bv),
            full(off1_w.T), full(off1_b[None]), full(w2d), full(b2d),
            full(p1), full(p1b), full(p2), full(p2b),
            full(wp), full(bp),
        ],
        out_specs=pl.BlockSpec((1, T, C), lambda b, t: (b, t, 0)),
        out_shape=jax.ShapeDtypeStruct((B, N, C), jnp.float32),
        scratch_shapes=[
            pltpu.VMEM((N, C), jnp.bfloat16),
            pltpu.VMEM((N, C), jnp.bfloat16),
        ],
        compiler_params=pltpu.CompilerParams(
            dimension_semantics=("arbitrary", "arbitrary"),
        ),
    )(coords, coordsT, x,
      wq, bq, wk, bk, wv, bv,
      off1_w.T, off1_b[None], w2d, b2d,
      p1, p1b, p2, p2b,
      wp, bp)
    return out
